# Initial kernel scaffold; baseline (speedup 1.0000x reference)
import jax
import jax.numpy as jnp
from jax.experimental import pallas as pl


def kernel(indices, data_table, unknown_table, W, b):
    raise NotImplementedError("write your pallas kernel here")


# speedup vs baseline: 4.0153x; 4.0153x over previous
import functools

import jax
import jax.numpy as jnp
from jax import lax
from jax.experimental import pallas as pl
from jax.experimental.pallas import tpu as pltpu
from jax.experimental.pallas import tpu_sc as plsc

_NUM_CORES = 2
_NUM_SUBCORES = 16
_NW = _NUM_CORES * _NUM_SUBCORES
_LANES = 16

_CHUNK = 1024
_GRP = 128


def _transform_table(data_table, W, b2d, unknown_table, blk):
    V, D = data_table.shape
    O = W.shape[0]
    grid = V // blk

    def body(x_ref, w_ref, b_ref, unk_ref, out_ref):
        t = lax.dot_general(
            x_ref[...], w_ref[...],
            (((1,), (1,)), ((), ())),
            preferred_element_type=jnp.float32,
        )
        out_ref[...] = t + b_ref[...]

        @pl.when(pl.program_id(0) == grid - 1)
        def _():
            out_ref[blk - 1, :] = unk_ref[0, :]

    return pl.pallas_call(
        body,
        grid=(grid,),
        in_specs=[
            pl.BlockSpec((blk, D), lambda i: (i, 0)),
            pl.BlockSpec((O, D), lambda i: (0, 0)),
            pl.BlockSpec((1, O), lambda i: (0, 0)),
            pl.BlockSpec((1, O), lambda i: (0, 0)),
        ],
        out_specs=pl.BlockSpec((blk, O), lambda i: (i, 0)),
        out_shape=jax.ShapeDtypeStruct((V, O), jnp.float32),
    )(data_table, W, b2d, unknown_table)


@functools.lru_cache(maxsize=None)
def _make_gather(V, O, N):
    per_w = N // _NW
    n_chunks = per_w // _CHUNK
    mesh = plsc.VectorSubcoreMesh(core_axis_name="c", subcore_axis_name="s")

    @functools.partial(
        pl.kernel,
        mesh=mesh,
        compiler_params=pltpu.CompilerParams(use_tc_tiling_on_sc=False),
        out_type=jax.ShapeDtypeStruct((N, O), jnp.float32),
        scratch_types=[
            pltpu.VMEM((_CHUNK,), jnp.int32),
            pltpu.VMEM((_CHUNK // _GRP, _GRP), jnp.int32),
            pltpu.VMEM((_CHUNK, O), jnp.float32),
            pltpu.SemaphoreType.DMA,
        ],
    )
    def gather_kernel(tbl_hbm, idx_hbm, out_hbm, idx_raw, idx_map, rows, sem):
        wid = lax.axis_index("s") * _NUM_CORES + lax.axis_index("c")
        base0 = wid * per_w

        def chunk_body(ci, carry):
            base = base0 + ci * _CHUNK
            pltpu.sync_copy(idx_hbm.at[pl.ds(base, _CHUNK)], idx_raw)
            for i in range(_CHUNK // _LANES):
                v = idx_raw[pl.ds(i * _LANES, _LANES)]
                j, k = divmod(i * _LANES, _GRP)
                idx_map[j, pl.ds(k, _LANES)] = jnp.where(v < 1, V - 1, v - 1)
            copies = [
                pltpu.async_copy(
                    tbl_hbm.at[idx_map.at[g]],
                    rows.at[pl.ds(g * _GRP, _GRP)],
                    sem,
                )
                for g in range(_CHUNK // _GRP)
            ]
            for c in copies:
                c.wait()
            pltpu.sync_copy(rows, out_hbm.at[pl.ds(base, _CHUNK)])
            return carry

        lax.fori_loop(0, n_chunks, chunk_body, 0)

    return gather_kernel


def kernel(indices, data_table, unknown_table, W, b):
    V, D = data_table.shape
    O = W.shape[0]
    B, H = indices.shape
    N = B * H

    T = _transform_table(
        data_table, W, b.reshape(1, O), unknown_table, blk=5000
    )
    idx_flat = indices.reshape(N).astype(jnp.int32)
    out = _make_gather(V, O, N)(T, idx_flat)
    return out.reshape(B, H, O)
